# baseline (speedup 1.0000x reference)
import functools

import jax
import jax.numpy as jnp
import numpy as np
from jax import lax
from jax.experimental import pallas as pl
from jax.experimental.pallas import tpu as pltpu
from jax.experimental.pallas import tpu_sc as plsc

N_SUB = 10000
E = 320000
D = 128
H = 4
DH = 32
N_ET = 10
D_ATTR = 16

_NB = 1000
_EB = 2000
_SCALE = 1.0 / np.sqrt(DH)
_CLAMP = 70.0

_NW = 32
_C = 128
_KE = 80
_EP = _NW * _KE * _C
_KN = 3
_NP = _NW * _KN * _C
_NACC = 10112
_DUMP = 10000
_RPT = _NACC // 16


def _head_sum_mat():
    i = lax.broadcasted_iota(jnp.int32, (D, H), 0)
    j = lax.broadcasted_iota(jnp.int32, (D, H), 1)
    return ((i // DH) == j).astype(jnp.float32)


def _head_expand_mat(rows):
    i = lax.broadcasted_iota(jnp.int32, (rows, D), 0)
    j = lax.broadcasted_iota(jnp.int32, (rows, D), 1)
    return (i == (j // DH)).astype(jnp.float32)


def _node_body(xf_ref, wemb_ref, bemb_ref, wq_ref, wk_ref, wv_ref,
               x_ref, xq_ref, xkv_ref):
    x = jnp.maximum(
        jnp.dot(xf_ref[...], wemb_ref[...], preferred_element_type=jnp.float32)
        + bemb_ref[...], 0.0)
    x_ref[...] = x
    xq_ref[...] = jnp.dot(x, wq_ref[...], preferred_element_type=jnp.float32)
    xkv_ref[:, :D] = jnp.dot(x, wk_ref[...], preferred_element_type=jnp.float32)
    xkv_ref[:, D:] = jnp.dot(x, wv_ref[...], preferred_element_type=jnp.float32)


def _node_pass(xf, W_emb, b_emb, Wq, Wk, Wv):
    grid = (N_SUB // _NB,)
    bspec = pl.BlockSpec((_NB, D), lambda i: (i, 0))
    wspec = pl.BlockSpec((D, D), lambda i: (0, 0))
    out = jax.ShapeDtypeStruct((N_SUB, D), jnp.float32)
    return pl.pallas_call(
        _node_body,
        grid=grid,
        in_specs=[bspec, wspec, pl.BlockSpec((1, D), lambda i: (0, 0)),
                  wspec, wspec, wspec],
        out_specs=[bspec, bspec, pl.BlockSpec((_NB, 2 * D), lambda i: (i, 0))],
        out_shape=[out, out, jax.ShapeDtypeStruct((N_SUB, 2 * D), jnp.float32)],
    )(xf, W_emb, b_emb.reshape(1, D), Wq, Wk, Wv)


def _edge_body(attr_ref, dt_ref, ds_ref, et_ref, kvsrc_ref, qdst_ref,
               wattr_ref, battr_ref, ttab_ref, tw_ref, tb_ref, dw_ref, db_ref,
               wk_ref, wv_ref, y_ref, ex_ref):
    attr = attr_ref[...]
    dt = dt_ref[...]
    ds = ds_ref[...]
    et = et_ref[...]
    ea = jnp.dot(attr, wattr_ref[...], preferred_element_type=jnp.float32) \
        + battr_ref[...]
    onehot = (lax.broadcasted_iota(jnp.int32, (attr.shape[0], 16), 1)
              == et).astype(jnp.float32)
    etyp = jnp.dot(onehot, ttab_ref[...], preferred_element_type=jnp.float32)
    tt = jnp.cos(dt * tw_ref[...] + tb_ref[...])
    ss = jnp.cos(ds * dw_ref[...] + db_ref[...])
    e = ea + etyp + tt + ss
    k = kvsrc_ref[:, :D] + jnp.dot(e, wk_ref[...],
                                   preferred_element_type=jnp.float32)
    v = kvsrc_ref[:, D:] + jnp.dot(e, wv_ref[...],
                                   preferred_element_type=jnp.float32)
    logits = jnp.dot(qdst_ref[...] * k, _head_sum_mat(),
                     preferred_element_type=jnp.float32) * _SCALE
    ex = jnp.exp(jnp.minimum(logits, _CLAMP))
    exb = jnp.dot(ex, _head_expand_mat(H), preferred_element_type=jnp.float32)
    y_ref[...] = v * exb
    ex_ref[...] = exb


def _edge_pass(edge_attr, dt, ds, et, kvsrc, qdst,
               W_attr, b_attr, ttab16, time_w, time_b, dist_w, dist_b, Wk, Wv):
    grid = (E // _EB,)
    eb256 = pl.BlockSpec((_EB, 2 * D), lambda i: (i, 0))
    eb128 = pl.BlockSpec((_EB, D), lambda i: (i, 0))
    eb1 = pl.BlockSpec((_EB, 1), lambda i: (i, 0))
    row = lambda d: pl.BlockSpec((1, d), lambda i: (0, 0))
    return pl.pallas_call(
        _edge_body,
        grid=grid,
        in_specs=[pl.BlockSpec((_EB, D_ATTR), lambda i: (i, 0)),
                  eb1, eb1, eb1, eb256, eb128,
                  pl.BlockSpec((D_ATTR, D), lambda i: (0, 0)),
                  row(D),
                  pl.BlockSpec((16, D), lambda i: (0, 0)),
                  row(D), row(D), row(D), row(D),
                  pl.BlockSpec((D, D), lambda i: (0, 0)),
                  pl.BlockSpec((D, D), lambda i: (0, 0))],
        out_specs=[eb128, eb128],
        out_shape=[jax.ShapeDtypeStruct((_EP, D), jnp.float32),
                   jax.ShapeDtypeStruct((_EP, D), jnp.float32)],
    )(edge_attr, dt, ds, et, kvsrc, qdst,
      W_attr, b_attr.reshape(1, D), ttab16,
      time_w.reshape(1, D), time_b.reshape(1, D),
      dist_w.reshape(1, D), dist_b.reshape(1, D), Wk, Wv)


def _final_body(agg0_ref, agg1_ref, den0_ref, den1_ref, x_ref, wo_ref,
                beta_ref, out_ref):
    denb = (den0_ref[...] + den1_ref[...])[0] + 1e-16
    a = (agg0_ref[...] + agg1_ref[...])[0] / denb
    out = jnp.dot(a, wo_ref[...], preferred_element_type=jnp.float32)
    g = jax.nn.sigmoid(beta_ref[0, 0])
    out_ref[...] = jnp.maximum(g * x_ref[...] + (1.0 - g) * out, 0.0)


def _final_pass(aggp, denp, x, Wo, beta):
    grid = (N_SUB // _NB,)
    bspec = pl.BlockSpec((_NB, D), lambda i: (i, 0))
    a0 = pl.BlockSpec((1, _NB, D), lambda i: (0, i, 0))
    a1 = pl.BlockSpec((1, _NB, D), lambda i: (1, i, 0))
    d0 = a0
    d1 = a1
    return pl.pallas_call(
        _final_body,
        grid=grid,
        in_specs=[a0, a1, d0, d1, bspec,
                  pl.BlockSpec((D, D), lambda i: (0, 0)),
                  pl.BlockSpec((1, 1), lambda i: (0, 0))],
        out_specs=bspec,
        out_shape=jax.ShapeDtypeStruct((N_SUB, D), jnp.float32),
    )(aggp, aggp, denp, denp, x, Wo, beta.reshape(1, 1))


def _sc_gather_features(features, nid_pad):
    mesh = plsc.VectorSubcoreMesh(core_axis_name="c", subcore_axis_name="s")

    @functools.partial(
        pl.kernel, mesh=mesh,
        out_type=jax.ShapeDtypeStruct((_NP, D), jnp.float32),
        scratch_types=[
            pltpu.VMEM((_KN, _C), jnp.int32),
            pltpu.VMEM((_C, D), jnp.float32),
            pltpu.SemaphoreType.DMA,
        ],
    )
    def k(feat_hbm, idx_hbm, out_hbm, ibuf, rbuf, sem):
        c = lax.axis_index("c")
        s = lax.axis_index("s")
        w = c * 16 + s
        pltpu.sync_copy(idx_hbm.at[w], ibuf)
        base = w * (_KN * _C)
        for j in range(_KN):
            pltpu.async_copy(feat_hbm.at[ibuf.at[j]], rbuf, sem).wait()
            pltpu.sync_copy(rbuf, out_hbm.at[pl.ds(base + j * _C, _C)])

    return k(features, nid_pad)


def _sc_gather_edges(xkv, xq, src_pad, dstg_pad):
    mesh = plsc.VectorSubcoreMesh(core_axis_name="c", subcore_axis_name="s")

    @functools.partial(
        pl.kernel, mesh=mesh,
        out_type=[jax.ShapeDtypeStruct((_EP, 2 * D), jnp.float32),
                  jax.ShapeDtypeStruct((_EP, D), jnp.float32)],
        scratch_types=[
            pltpu.VMEM((_KE, _C), jnp.int32),
            pltpu.VMEM((_KE, _C), jnp.int32),
            pltpu.VMEM((_C, 2 * D), jnp.float32),
            pltpu.VMEM((_C, 2 * D), jnp.float32),
            pltpu.VMEM((_C, D), jnp.float32),
            pltpu.VMEM((_C, D), jnp.float32),
            pltpu.SemaphoreType.DMA,
            pltpu.SemaphoreType.DMA,
            pltpu.SemaphoreType.DMA,
            pltpu.SemaphoreType.DMA,
        ],
    )
    def k(xkv_hbm, xq_hbm, src_hbm, dst_hbm, kvo, qo,
          sbuf, dbuf, kvb0, kvb1, qb0, qb1, gs0, gs1, ws0, ws1):
        c = lax.axis_index("c")
        s = lax.axis_index("s")
        w = c * 16 + s
        pltpu.sync_copy(src_hbm.at[w], sbuf)
        pltpu.sync_copy(dst_hbm.at[w], dbuf)
        base = w * (_KE * _C)
        slots = ((kvb0, qb0, gs0, ws0), (kvb1, qb1, gs1, ws1))

        def start_gather(j, kvb, qb, gs):
            pltpu.async_copy(xkv_hbm.at[sbuf.at[j]], kvb, gs)
            pltpu.async_copy(xq_hbm.at[dbuf.at[j]], qb, gs)

        start_gather(0, kvb0, qb0, gs0)
        start_gather(1, kvb1, qb1, gs1)

        def body(t, carry):
            for b in range(2):
                kvb, qb, gs, ws = slots[b]
                j = 2 * t + b
                pltpu.make_async_copy(xkv_hbm.at[pl.ds(0, _C)], kvb, gs).wait()
                pltpu.make_async_copy(xq_hbm.at[pl.ds(0, _C)], qb, gs).wait()
                off = base + j * _C
                w1 = pltpu.async_copy(kvb, kvo.at[pl.ds(off, _C)], ws)
                w2 = pltpu.async_copy(qb, qo.at[pl.ds(off, _C)], ws)
                w1.wait()
                w2.wait()

                @pl.when(j + 2 < _KE)
                def _():
                    start_gather(j + 2, kvb, qb, gs)
            return carry

        lax.fori_loop(0, _KE // 2, body, 0)

    return k(xkv, xq, src_pad, dstg_pad)


def _sc_scatter(vals, dsts_pad, zeros, width):
    mesh = plsc.VectorSubcoreMesh(core_axis_name="c", subcore_axis_name="s")

    @functools.partial(
        pl.kernel, mesh=mesh,
        out_type=jax.ShapeDtypeStruct((2, _NACC, width), jnp.float32),
        scratch_types=[
            pltpu.VMEM((_KE, _C), jnp.int32),
            pltpu.VMEM((_C, width), jnp.float32),
            pltpu.VMEM_SHARED((_NACC, width), jnp.float32),
        ],
    )
    def k(v_hbm, dst_hbm, z_hbm, accp, ibuf, vbuf, acc_sh):
        c = lax.axis_index("c")
        s = lax.axis_index("s")
        w = c * 16 + s
        pltpu.sync_copy(z_hbm, acc_sh.at[pl.ds(s * _RPT, _RPT)])
        plsc.subcore_barrier()
        pltpu.sync_copy(dst_hbm.at[w], ibuf)
        base = w * (_KE * _C)

        def body(j, carry):
            off = base + j * _C
            pltpu.sync_copy(v_hbm.at[pl.ds(off, _C)], vbuf)
            pltpu.sync_copy(vbuf, acc_sh.at[ibuf.at[j]], add=True)
            return carry

        lax.fori_loop(0, _KE, body, 0)
        plsc.subcore_barrier()
        pltpu.sync_copy(acc_sh.at[pl.ds(s * _RPT, _RPT)],
                        accp.at[c].at[pl.ds(s * _RPT, _RPT)])

    return k(vals, dsts_pad, zeros)


def kernel(edge_attr, edge_delta_t, edge_delta_s, features, W_emb, b_emb,
           W_attr, b_attr, type_table, time_w, time_b, dist_w, dist_b,
           Wq, Wk, Wv, Wo, beta, n_id, edge_index, edge_type):
    src = edge_index[0].astype(jnp.int32)
    dst = edge_index[1].astype(jnp.int32)

    nid_pad = jnp.concatenate(
        [n_id.astype(jnp.int32),
         jnp.zeros((_NP - N_SUB,), jnp.int32)]).reshape(_NW, _KN, _C)
    src_pad = jnp.concatenate(
        [src, jnp.zeros((_EP - E,), jnp.int32)]).reshape(_NW, _KE, _C)
    dstg_pad = jnp.concatenate(
        [dst, jnp.zeros((_EP - E,), jnp.int32)]).reshape(_NW, _KE, _C)
    dsts_pad = jnp.concatenate(
        [dst, jnp.full((_EP - E,), _DUMP, jnp.int32)]).reshape(_NW, _KE, _C)

    xf = _sc_gather_features(features, nid_pad)

    x, xq, xkv = _node_pass(xf, W_emb, b_emb, Wq, Wk, Wv)

    kvsrc, qdst = _sc_gather_edges(xkv, xq, src_pad, dstg_pad)

    ttab16 = jnp.concatenate(
        [type_table, jnp.zeros((16 - N_ET, D), jnp.float32)], axis=0)
    y, ex = _edge_pass(edge_attr, edge_delta_t.reshape(E, 1),
                       edge_delta_s.reshape(E, 1),
                       edge_type.reshape(E, 1).astype(jnp.int32),
                       kvsrc, qdst,
                       W_attr, b_attr, ttab16, time_w, time_b, dist_w, dist_b,
                       Wk, Wv)

    zrows = jnp.zeros((_RPT, D), jnp.float32)
    aggp = _sc_scatter(y, dsts_pad, zrows, D)
    denp = _sc_scatter(ex, dsts_pad, zrows, D)

    return _final_pass(aggp, denp, x, Wo, beta)


# speedup vs baseline: 1.0711x; 1.0711x over previous
import functools

import jax
import jax.numpy as jnp
import numpy as np
from jax import lax
from jax.experimental import pallas as pl
from jax.experimental.pallas import tpu as pltpu
from jax.experimental.pallas import tpu_sc as plsc

N_SUB = 10000
E = 320000
D = 128
H = 4
DH = 32
N_ET = 10
D_ATTR = 16

_NB = 1000
_EB = 2000
_SCALE = 1.0 / np.sqrt(DH)
_CLAMP = 70.0

_NW = 32
_C = 128
_KE = 79
_EP = _NW * _KE * _C
_KN = 3
_NP = _NW * _KN * _C
_NACC = 10112
_DUMP = 10000
_RPT = _NACC // 16


def _head_sum_mat():
    i = lax.broadcasted_iota(jnp.int32, (D, H), 0)
    j = lax.broadcasted_iota(jnp.int32, (D, H), 1)
    return ((i // DH) == j).astype(jnp.float32)


def _head_expand_mat(rows):
    i = lax.broadcasted_iota(jnp.int32, (rows, D), 0)
    j = lax.broadcasted_iota(jnp.int32, (rows, D), 1)
    return (i == (j // DH)).astype(jnp.float32)


def _node_body(xf_ref, wemb_ref, bemb_ref, wq_ref, wk_ref, wv_ref,
               x_ref, xq_ref, xk_ref, xv_ref):
    x = jnp.maximum(
        jnp.dot(xf_ref[...], wemb_ref[...], preferred_element_type=jnp.float32)
        + bemb_ref[...], 0.0)
    x_ref[...] = x
    xq_ref[...] = jnp.dot(x, wq_ref[...], preferred_element_type=jnp.float32)
    xk_ref[...] = jnp.dot(x, wk_ref[...], preferred_element_type=jnp.float32)
    xv_ref[...] = jnp.dot(x, wv_ref[...], preferred_element_type=jnp.float32)


def _node_pass(xf, W_emb, b_emb, Wq, Wk, Wv):
    grid = (N_SUB // _NB,)
    bspec = pl.BlockSpec((_NB, D), lambda i: (i, 0))
    wspec = pl.BlockSpec((D, D), lambda i: (0, 0))
    out = jax.ShapeDtypeStruct((N_SUB, D), jnp.float32)
    return pl.pallas_call(
        _node_body,
        grid=grid,
        in_specs=[bspec, wspec, pl.BlockSpec((1, D), lambda i: (0, 0)),
                  wspec, wspec, wspec],
        out_specs=[bspec, bspec, bspec, bspec],
        out_shape=[out, out, out, out],
    )(xf, W_emb, b_emb.reshape(1, D), Wq, Wk, Wv)


def _edge_body(attr_ref, dt_ref, ds_ref, et_ref, ksrc_ref, qdst_ref, vsrc_ref,
               wattr_ref, battr_ref, ttab_ref, tw_ref, tb_ref, dw_ref, db_ref,
               wk_ref, wv_ref, y_ref, ex_ref):
    attr = attr_ref[...]
    dt = dt_ref[...]
    ds = ds_ref[...]
    et = et_ref[...]
    ea = jnp.dot(attr, wattr_ref[...], preferred_element_type=jnp.float32) \
        + battr_ref[...]
    onehot = (lax.broadcasted_iota(jnp.int32, (attr.shape[0], 16), 1)
              == et).astype(jnp.float32)
    etyp = jnp.dot(onehot, ttab_ref[...], preferred_element_type=jnp.float32)
    tt = jnp.cos(dt * tw_ref[...] + tb_ref[...])
    ss = jnp.cos(ds * dw_ref[...] + db_ref[...])
    e = ea + etyp + tt + ss
    k = ksrc_ref[...] + jnp.dot(e, wk_ref[...], preferred_element_type=jnp.float32)
    v = vsrc_ref[...] + jnp.dot(e, wv_ref[...], preferred_element_type=jnp.float32)
    logits = jnp.dot(qdst_ref[...] * k, _head_sum_mat(),
                     preferred_element_type=jnp.float32) * _SCALE
    ex = jnp.exp(jnp.minimum(logits, _CLAMP))
    exb = jnp.dot(ex, _head_expand_mat(H), preferred_element_type=jnp.float32)
    y_ref[...] = v * exb
    ex_ref[...] = exb


def _edge_pass(edge_attr, dt, ds, et, ksrc, qdst, vsrc,
               W_attr, b_attr, ttab16, time_w, time_b, dist_w, dist_b, Wk, Wv):
    grid = (E // _EB,)
    eb128 = pl.BlockSpec((_EB, D), lambda i: (i, 0))
    eb1 = pl.BlockSpec((_EB, 1), lambda i: (i, 0))
    row = lambda d: pl.BlockSpec((1, d), lambda i: (0, 0))
    return pl.pallas_call(
        _edge_body,
        grid=grid,
        in_specs=[pl.BlockSpec((_EB, D_ATTR), lambda i: (i, 0)),
                  eb1, eb1, eb1, eb128, eb128, eb128,
                  pl.BlockSpec((D_ATTR, D), lambda i: (0, 0)),
                  row(D),
                  pl.BlockSpec((16, D), lambda i: (0, 0)),
                  row(D), row(D), row(D), row(D),
                  pl.BlockSpec((D, D), lambda i: (0, 0)),
                  pl.BlockSpec((D, D), lambda i: (0, 0))],
        out_specs=[eb128, eb128],
        out_shape=[jax.ShapeDtypeStruct((_EP, D), jnp.float32),
                   jax.ShapeDtypeStruct((_EP, D), jnp.float32)],
    )(edge_attr, dt, ds, et, ksrc, qdst, vsrc,
      W_attr, b_attr.reshape(1, D), ttab16,
      time_w.reshape(1, D), time_b.reshape(1, D),
      dist_w.reshape(1, D), dist_b.reshape(1, D), Wk, Wv)


def _final_body(agg0_ref, agg1_ref, den0_ref, den1_ref, x_ref, wo_ref,
                beta_ref, out_ref):
    denb = (den0_ref[...] + den1_ref[...])[0] + 1e-16
    a = (agg0_ref[...] + agg1_ref[...])[0] / denb
    out = jnp.dot(a, wo_ref[...], preferred_element_type=jnp.float32)
    g = jax.nn.sigmoid(beta_ref[0, 0])
    out_ref[...] = jnp.maximum(g * x_ref[...] + (1.0 - g) * out, 0.0)


def _final_pass(aggp, denp, x, Wo, beta):
    grid = (N_SUB // _NB,)
    bspec = pl.BlockSpec((_NB, D), lambda i: (i, 0))
    a0 = pl.BlockSpec((1, _NB, D), lambda i: (0, i, 0))
    a1 = pl.BlockSpec((1, _NB, D), lambda i: (1, i, 0))
    d0 = a0
    d1 = a1
    return pl.pallas_call(
        _final_body,
        grid=grid,
        in_specs=[a0, a1, d0, d1, bspec,
                  pl.BlockSpec((D, D), lambda i: (0, 0)),
                  pl.BlockSpec((1, 1), lambda i: (0, 0))],
        out_specs=bspec,
        out_shape=jax.ShapeDtypeStruct((N_SUB, D), jnp.float32),
    )(aggp, aggp, denp, denp, x, Wo, beta.reshape(1, 1))


def _sc_gather_features(features, nid_pad):
    mesh = plsc.VectorSubcoreMesh(core_axis_name="c", subcore_axis_name="s")

    @functools.partial(
        pl.kernel, mesh=mesh,
        out_type=jax.ShapeDtypeStruct((_NP, D), jnp.float32),
        scratch_types=[
            pltpu.VMEM((_KN, _C), jnp.int32),
            pltpu.VMEM((_C, D), jnp.float32),
            pltpu.SemaphoreType.DMA,
        ],
    )
    def k(feat_hbm, idx_hbm, out_hbm, ibuf, rbuf, sem):
        c = lax.axis_index("c")
        s = lax.axis_index("s")
        w = c * 16 + s
        pltpu.sync_copy(idx_hbm.at[w], ibuf)
        base = w * (_KN * _C)
        for j in range(_KN):
            pltpu.async_copy(feat_hbm.at[ibuf.at[j]], rbuf, sem).wait()
            pltpu.sync_copy(rbuf, out_hbm.at[pl.ds(base + j * _C, _C)])

    return k(features, nid_pad)


def _sc_gather_edges(xk, xq, xv, src_pad, dstg_pad):
    mesh = plsc.VectorSubcoreMesh(core_axis_name="c", subcore_axis_name="s")
    outt = jax.ShapeDtypeStruct((_EP, D), jnp.float32)

    @functools.partial(
        pl.kernel, mesh=mesh,
        out_type=[outt, outt, outt],
        scratch_types=[
            pltpu.VMEM((_KE, _C), jnp.int32),
            pltpu.VMEM((_KE, _C), jnp.int32),
            pltpu.VMEM((_C, D), jnp.float32),
            pltpu.VMEM((_C, D), jnp.float32),
            pltpu.VMEM((_C, D), jnp.float32),
            pltpu.SemaphoreType.DMA,
            pltpu.SemaphoreType.DMA,
            pltpu.SemaphoreType.DMA,
        ],
    )
    def k(xk_hbm, xq_hbm, xv_hbm, src_hbm, dst_hbm, ko, qo, vo,
          sbuf, dbuf, kb, qb, vb, sem1, sem2, sem3):
        c = lax.axis_index("c")
        s = lax.axis_index("s")
        w = c * 16 + s
        pltpu.sync_copy(src_hbm.at[w], sbuf)
        pltpu.sync_copy(dst_hbm.at[w], dbuf)
        base = w * (_KE * _C)

        def body(j, carry):
            cp1 = pltpu.async_copy(xk_hbm.at[sbuf.at[j]], kb, sem1)
            cp2 = pltpu.async_copy(xq_hbm.at[dbuf.at[j]], qb, sem2)
            cp3 = pltpu.async_copy(xv_hbm.at[sbuf.at[j]], vb, sem3)
            cp1.wait()
            cp2.wait()
            cp3.wait()
            off = base + j * _C
            pltpu.sync_copy(kb, ko.at[pl.ds(off, _C)])
            pltpu.sync_copy(qb, qo.at[pl.ds(off, _C)])
            pltpu.sync_copy(vb, vo.at[pl.ds(off, _C)])
            return carry

        lax.fori_loop(0, _KE, body, 0)

    return k(xk, xq, xv, src_pad, dstg_pad)


def _sc_scatter(vals, dsts_pad, zeros, width):
    mesh = plsc.VectorSubcoreMesh(core_axis_name="c", subcore_axis_name="s")

    @functools.partial(
        pl.kernel, mesh=mesh,
        out_type=jax.ShapeDtypeStruct((2, _NACC, width), jnp.float32),
        scratch_types=[
            pltpu.VMEM((_KE, _C), jnp.int32),
            pltpu.VMEM((_C, width), jnp.float32),
            pltpu.VMEM_SHARED((_NACC, width), jnp.float32),
        ],
    )
    def k(v_hbm, dst_hbm, z_hbm, accp, ibuf, vbuf, acc_sh):
        c = lax.axis_index("c")
        s = lax.axis_index("s")
        w = c * 16 + s
        pltpu.sync_copy(z_hbm, acc_sh.at[pl.ds(s * _RPT, _RPT)])
        plsc.subcore_barrier()
        pltpu.sync_copy(dst_hbm.at[w], ibuf)
        base = w * (_KE * _C)

        def body(j, carry):
            off = base + j * _C
            pltpu.sync_copy(v_hbm.at[pl.ds(off, _C)], vbuf)
            pltpu.sync_copy(vbuf, acc_sh.at[ibuf.at[j]], add=True)
            return carry

        lax.fori_loop(0, _KE, body, 0)
        plsc.subcore_barrier()
        pltpu.sync_copy(acc_sh.at[pl.ds(s * _RPT, _RPT)],
                        accp.at[c].at[pl.ds(s * _RPT, _RPT)])

    return k(vals, dsts_pad, zeros)


def kernel(edge_attr, edge_delta_t, edge_delta_s, features, W_emb, b_emb,
           W_attr, b_attr, type_table, time_w, time_b, dist_w, dist_b,
           Wq, Wk, Wv, Wo, beta, n_id, edge_index, edge_type):
    src = edge_index[0].astype(jnp.int32)
    dst = edge_index[1].astype(jnp.int32)

    nid_pad = jnp.concatenate(
        [n_id.astype(jnp.int32),
         jnp.zeros((_NP - N_SUB,), jnp.int32)]).reshape(_NW, _KN, _C)
    src_pad = jnp.concatenate(
        [src, jnp.zeros((_EP - E,), jnp.int32)]).reshape(_NW, _KE, _C)
    dstg_pad = jnp.concatenate(
        [dst, jnp.zeros((_EP - E,), jnp.int32)]).reshape(_NW, _KE, _C)
    dsts_pad = jnp.concatenate(
        [dst, jnp.full((_EP - E,), _DUMP, jnp.int32)]).reshape(_NW, _KE, _C)

    xf = _sc_gather_features(features, nid_pad)

    x, xq, xk, xv = _node_pass(xf, W_emb, b_emb, Wq, Wk, Wv)

    ksrc, qdst, vsrc = _sc_gather_edges(xk, xq, xv, src_pad, dstg_pad)

    ttab16 = jnp.concatenate(
        [type_table, jnp.zeros((16 - N_ET, D), jnp.float32)], axis=0)
    y, ex = _edge_pass(edge_attr, edge_delta_t.reshape(E, 1),
                       edge_delta_s.reshape(E, 1),
                       edge_type.reshape(E, 1).astype(jnp.int32),
                       ksrc, qdst, vsrc,
                       W_attr, b_attr, ttab16, time_w, time_b, dist_w, dist_b,
                       Wk, Wv)

    zrows = jnp.zeros((_RPT, D), jnp.float32)
    aggp = _sc_scatter(y, dsts_pad, zrows, D)
    denp = _sc_scatter(ex, dsts_pad, zrows, D)

    return _final_pass(aggp, denp, x, Wo, beta)
